# baseline (speedup 1.0000x reference)
import jax
import jax.numpy as jnp
from jax import lax
from jax.experimental import pallas as pl
from jax.experimental.pallas import tpu as pltpu
from jax.experimental.pallas import tpu_sc as plsc

_B, _T, _N, _D = 8, 16, 196, 768
_RVECS = _D // 16
_GROWS = 48
_PROWS = 24
_NTAIL = _N - 4 * _GROWS
_NB = 8
_LEAD = 4


def _sc_body(x_hbm, tbl_hbm, out_hbm, tbl_v, bufs, sins, souts):
    wid = lax.axis_index("c") * 16 + lax.axis_index("s")
    b = wid // 4
    g = lax.rem(wid, 4)
    n0 = _GROWS * g

    def add_row(v, row):
        @plsc.parallel_loop(0, _RVECS, 1, unroll=2)
        def add(i):
            tv = tbl_v[row, pl.ds(i * 16, 16)]
            for t in range(_T):
                plsc.addupdate(v.at[t, pl.ds(i * 16, 16)], tv)

    def phase(p0):
        pltpu.sync_copy(
            tbl_hbm.at[pl.ds(n0 + p0, _PROWS), :], tbl_v
        )

        for r in range(_LEAD):
            pltpu.async_copy(x_hbm.at[b, n0 + p0 + r], bufs[r], sins[r])

        @pl.loop(0, _PROWS, step=_NB)
        def jobs(jbase):
            for r in range(_NB):
                j = jbase + r
                v, s_in, s_out = bufs[r], sins[r], souts[r]
                pltpu.make_async_copy(x_hbm.at[b, n0 + p0 + j], v, s_in).wait()
                add_row(v, j)
                pltpu.async_copy(v, out_hbm.at[b, n0 + p0 + j], s_out)

                jn = j + _LEAD
                rn = (r + _LEAD) % _NB

                @pl.when(jn < _PROWS)
                def _():
                    @pl.when(jn >= _NB)
                    def _():
                        pltpu.make_async_copy(
                            bufs[rn], out_hbm.at[b, n0 + p0 + jn - _NB], souts[rn]
                        ).wait()

                    pltpu.async_copy(
                        x_hbm.at[b, n0 + p0 + jn], bufs[rn], sins[rn]
                    )

        for r in range(_NB):
            j = _PROWS - _NB + r
            pltpu.make_async_copy(
                bufs[r], out_hbm.at[b, n0 + p0 + j], souts[r]
            ).wait()

    phase(0)
    phase(_PROWS)

    pltpu.sync_copy(
        tbl_hbm.at[pl.ds(4 * _GROWS, _NTAIL), :], tbl_v.at[pl.ds(0, _NTAIL), :]
    )
    n_tail = 4 * _GROWS + g
    pltpu.sync_copy(x_hbm.at[b, n_tail], bufs[0])
    add_row(bufs[0], g)
    pltpu.sync_copy(bufs[0], out_hbm.at[b, n_tail])


def kernel(encoded_tokens, pos_table):
    B, T, N, D = encoded_tokens.shape
    xt = jnp.transpose(encoded_tokens, (0, 2, 1, 3))
    mesh = plsc.VectorSubcoreMesh(core_axis_name="c", subcore_axis_name="s")
    run = pl.kernel(
        _sc_body,
        mesh=mesh,
        out_type=jax.ShapeDtypeStruct((B, N, T, D), jnp.float32),
        scratch_types=[
            pltpu.VMEM((_PROWS, _D), jnp.float32),
            [pltpu.VMEM((_T, _D), jnp.float32) for _ in range(_NB)],
            [pltpu.SemaphoreType.DMA for _ in range(_NB)],
            [pltpu.SemaphoreType.DMA for _ in range(_NB)],
        ],
    )
    out_t = run(xt, pos_table)
    return jnp.transpose(out_t, (0, 2, 1, 3))


# speedup vs baseline: 1.0671x; 1.0671x over previous
import jax
import jax.numpy as jnp
from jax import lax
from jax.experimental import pallas as pl
from jax.experimental.pallas import tpu as pltpu
from jax.experimental.pallas import tpu_sc as plsc

_B, _T, _N, _D = 8, 16, 196, 768
_RVECS = _D // 16
_GROWS = 48
_NTAIL = _N - 4 * _GROWS
_NB = 6
_LEAD = 3


def _sc_body(x_hbm, tbl_hbm, out_hbm, tbl_v, tail_v, tail_t,
             bufs, sins, souts, s_tl, s_tt):
    wid = lax.axis_index("c") * 16 + lax.axis_index("s")
    b = wid // 4
    g = lax.rem(wid, 4)
    n0 = _GROWS * g

    def add_row(v, tref, row):
        @plsc.parallel_loop(0, _RVECS, 1, unroll=2)
        def add(i):
            tv = tref[row, pl.ds(i * 16, 16)]
            for t in range(_T):
                plsc.addupdate(v.at[t, pl.ds(i * 16, 16)], tv)

    pltpu.sync_copy(tbl_hbm.at[pl.ds(n0, _GROWS), :], tbl_v)

    n_tail = 4 * _GROWS + g
    pltpu.async_copy(x_hbm.at[b, n_tail], tail_v, s_tl)
    pltpu.async_copy(
        tbl_hbm.at[pl.ds(4 * _GROWS, _NTAIL), :], tail_t, s_tt
    )

    for r in range(_LEAD):
        pltpu.async_copy(x_hbm.at[b, n0 + r], bufs[r], sins[r])

    @pl.loop(0, _GROWS, step=_NB)
    def jobs(jbase):
        for r in range(_NB):
            j = jbase + r
            v, s_in, s_out = bufs[r], sins[r], souts[r]
            pltpu.make_async_copy(x_hbm.at[b, n0 + j], v, s_in).wait()
            add_row(v, tbl_v, j)
            pltpu.async_copy(v, out_hbm.at[b, n0 + j], s_out)

            jn = j + _LEAD
            rn = (r + _LEAD) % _NB

            @pl.when(jn < _GROWS)
            def _():
                @pl.when(jn >= _NB)
                def _():
                    pltpu.make_async_copy(
                        bufs[rn], out_hbm.at[b, n0 + jn - _NB], souts[rn]
                    ).wait()

                pltpu.async_copy(x_hbm.at[b, n0 + jn], bufs[rn], sins[rn])

    for r in range(_NB):
        j = _GROWS - _NB + r
        pltpu.make_async_copy(
            bufs[r], out_hbm.at[b, n0 + j], souts[r]
        ).wait()

    pltpu.make_async_copy(x_hbm.at[b, n_tail], tail_v, s_tl).wait()
    pltpu.make_async_copy(
        tbl_hbm.at[pl.ds(4 * _GROWS, _NTAIL), :], tail_t, s_tt
    ).wait()
    add_row(tail_v, tail_t, g)
    pltpu.sync_copy(tail_v, out_hbm.at[b, n_tail])


def kernel(encoded_tokens, pos_table):
    B, T, N, D = encoded_tokens.shape
    xt = jnp.transpose(encoded_tokens, (0, 2, 1, 3))
    mesh = plsc.VectorSubcoreMesh(core_axis_name="c", subcore_axis_name="s")
    run = pl.kernel(
        _sc_body,
        mesh=mesh,
        out_type=jax.ShapeDtypeStruct((B, N, T, D), jnp.float32),
        scratch_types=[
            pltpu.VMEM((_GROWS, _D), jnp.float32),
            pltpu.VMEM((_T, _D), jnp.float32),
            pltpu.VMEM((_NTAIL, _D), jnp.float32),
            [pltpu.VMEM((_T, _D), jnp.float32) for _ in range(_NB)],
            [pltpu.SemaphoreType.DMA for _ in range(_NB)],
            [pltpu.SemaphoreType.DMA for _ in range(_NB)],
            pltpu.SemaphoreType.DMA,
            pltpu.SemaphoreType.DMA,
        ],
    )
    out_t = run(xt, pos_table)
    return jnp.transpose(out_t, (0, 2, 1, 3))
